# Initial kernel scaffold; baseline (speedup 1.0000x reference)
import jax
import jax.numpy as jnp
from jax.experimental import pallas as pl


def kernel(x, edge_index, W1_l, b1, W1_r, W2_l, b2, W2_r, W3_l, b3, W3_r):
    raise NotImplementedError("write your pallas kernel here")


# speedup vs baseline: 2.1675x; 2.1675x over previous
import functools

import jax
import jax.numpy as jnp
from jax import lax
from jax.experimental import pallas as pl
from jax.experimental.pallas import tpu as pltpu
from jax.experimental.pallas import tpu_sc as plsc

N_NODES = 10000
N_EDGES = 160000
M_PAD = 10240
NS = 16
NC = 2
CH = 128
NCHUNK = 80
E_PAD = NS * NCHUNK * CH
RPT = M_PAD // NS
ZROWS = 320
SLAB = 64

_MESH = dict(core_axis_name="c", subcore_axis_name="s",
             num_cores=NC, num_subcores=NS)


def _fill(ref, rows, value):
    def _f(i, carry):
        for j in range(SLAB // 16):
            ref[i, pl.ds(j * 16, 16)] = jnp.full((16,), value, jnp.float32)
        return carry
    lax.fori_loop(0, rows, _f, 0)


def _deg_body(dst3_ref, deg_out, dstb, onesv, zerov, degacc):
    c = lax.axis_index("c")
    sid = lax.axis_index("s")
    pltpu.sync_copy(dst3_ref.at[sid], dstb)
    _fill(onesv, CH, 1.0)
    _fill(zerov, ZROWS, 0.0)
    for z in range(RPT // ZROWS):
        pltpu.sync_copy(zerov, degacc.at[pl.ds(sid * RPT + z * ZROWS, ZROWS)])
    plsc.subcore_barrier()

    def _edge(j, carry):
        pltpu.sync_copy(onesv, degacc.at[dstb.at[j]], add=True)
        return carry
    lax.fori_loop(0, NCHUNK, _edge, 0)
    plsc.subcore_barrier()

    @pl.when(c == 0)
    def _():
        pltpu.sync_copy(degacc.at[pl.ds(sid * RPT, RPT)],
                        deg_out.at[pl.ds(sid * RPT, RPT)])


def _sc_degree(dst3):
    call = pl.kernel(
        _deg_body,
        out_type=jax.ShapeDtypeStruct((M_PAD, SLAB), jnp.float32),
        mesh=plsc.VectorSubcoreMesh(**_MESH),
        compiler_params=pltpu.CompilerParams(use_tc_tiling_on_sc=False),
        scratch_types=(
            pltpu.VMEM((NCHUNK, CH), jnp.int32),
            pltpu.VMEM((CH, SLAB), jnp.float32),
            pltpu.VMEM((ZROWS, SLAB), jnp.float32),
            pltpu.VMEM_SHARED((M_PAD, SLAB), jnp.float32),
        ),
    )
    return call(dst3)


def _agg_body(S, h_ref, src3_ref, dst3_ref, agg_out,
              srcb, dstb, idxb, rowb, zerov, acc):
    c = lax.axis_index("c")
    sid = lax.axis_index("s")
    s_half = S // NC

    pltpu.sync_copy(src3_ref.at[sid], srcb)
    pltpu.sync_copy(dst3_ref.at[sid], dstb)
    _fill(zerov, ZROWS, 0.0)

    for js in range(s_half):
        s_glob = c * s_half + js
        for z in range(RPT // ZROWS):
            pltpu.sync_copy(zerov,
                            acc.at[pl.ds(sid * RPT + z * ZROWS, ZROWS)])
        plsc.subcore_barrier()

        def _ix(i, carry):
            for j in range(CH // 16):
                idxb[i, pl.ds(j * 16, 16)] = (
                    srcb[i, pl.ds(j * 16, 16)] * S + s_glob)
            return carry
        lax.fori_loop(0, NCHUNK, _ix, 0)

        def _edge(j, carry):
            pltpu.sync_copy(h_ref.at[idxb.at[j]], rowb)
            pltpu.sync_copy(rowb, acc.at[dstb.at[j]], add=True)
            return carry
        lax.fori_loop(0, NCHUNK, _edge, 0)
        plsc.subcore_barrier()

        pltpu.sync_copy(
            acc.at[pl.ds(sid * RPT, RPT)],
            agg_out.at[s_glob, pl.ds(sid * RPT, RPT)])
        if js != s_half - 1:
            plsc.subcore_barrier()


def _sc_aggregate(h, src3, dst3, S):
    call = pl.kernel(
        functools.partial(_agg_body, S),
        out_type=jax.ShapeDtypeStruct((S, M_PAD, SLAB), jnp.float32),
        mesh=plsc.VectorSubcoreMesh(**_MESH),
        compiler_params=pltpu.CompilerParams(use_tc_tiling_on_sc=False),
        scratch_types=(
            pltpu.VMEM((NCHUNK, CH), jnp.int32),
            pltpu.VMEM((NCHUNK, CH), jnp.int32),
            pltpu.VMEM((NCHUNK, CH), jnp.int32),
            pltpu.VMEM((CH, SLAB), jnp.float32),
            pltpu.VMEM((ZROWS, SLAB), jnp.float32),
            pltpu.VMEM_SHARED((M_PAD, SLAB), jnp.float32),
        ),
    )
    return call(h.reshape(M_PAD * S, SLAB), src3, dst3)


def _fused_body(deg_ref, agg_ref, x_ref, wl_ref, wr_ref, b_ref, out_ref,
                acc_ref):
    s = pl.program_id(2)
    ns = pl.num_programs(2)

    @pl.when(s == 0)
    def _():
        acc_ref[...] = jnp.zeros_like(acc_ref)

    acc_ref[...] += jnp.dot(agg_ref[0], wl_ref[0],
                            preferred_element_type=jnp.float32)

    @pl.when(s == ns - 1)
    def _():
        scale = 1.0 / jnp.maximum(deg_ref[...], 1.0)
        acc = acc_ref[...] * scale + jnp.dot(
            x_ref[...], wr_ref[...], preferred_element_type=jnp.float32)
        out_ref[...] = jnp.maximum(acc + b_ref[...], 0.0)


def _tc_fused(agg, x, wl, wr, b, deg, bm=512, bn=512):
    m, k = x.shape
    S = agg.shape[0]
    dout = wl.shape[1]
    bn = min(bn, dout)
    wl3 = wl.reshape(S, SLAB, dout)
    grid = (m // bm, dout // bn, S)
    return pl.pallas_call(
        _fused_body,
        grid=grid,
        in_specs=[
            pl.BlockSpec((bm, 1), lambda i, j, s: (i, 0)),
            pl.BlockSpec((1, bm, SLAB), lambda i, j, s: (s, i, 0)),
            pl.BlockSpec((bm, k), lambda i, j, s: (i, 0)),
            pl.BlockSpec((1, SLAB, bn), lambda i, j, s: (s, 0, j)),
            pl.BlockSpec((k, bn), lambda i, j, s: (0, j)),
            pl.BlockSpec((1, bn), lambda i, j, s: (0, j)),
        ],
        out_specs=pl.BlockSpec((bm, bn), lambda i, j, s: (i, j)),
        out_shape=jax.ShapeDtypeStruct((m, dout), jnp.float32),
        scratch_shapes=[pltpu.VMEM((bm, bn), jnp.float32)],
    )(deg, agg, x, wl3, wr, b)


def _mm_body(a_ref, w_ref, o_ref):
    o_ref[...] = jnp.dot(a_ref[...], w_ref[...],
                         preferred_element_type=jnp.float32)


def _tc_matmul(a, w, bm=512):
    m, k = a.shape
    dout = w.shape[1]
    return pl.pallas_call(
        _mm_body,
        grid=(m // bm,),
        in_specs=[
            pl.BlockSpec((bm, k), lambda i: (i, 0)),
            pl.BlockSpec((k, dout), lambda i: (0, 0)),
        ],
        out_specs=pl.BlockSpec((bm, dout), lambda i: (i, 0)),
        out_shape=jax.ShapeDtypeStruct((m, dout), jnp.float32),
    )(a, w)


def _final_body(S, deg_ref, agg_ref, h_ref, wr_ref, b_ref, out_ref):
    scale = 1.0 / jnp.maximum(deg_ref[...], 1.0)
    agg = jnp.concatenate([agg_ref[s] for s in range(S)], axis=1)
    acc = agg * scale + jnp.dot(h_ref[...], wr_ref[...],
                                preferred_element_type=jnp.float32)
    out_ref[...] = jnp.maximum(acc + b_ref[...], 0.0)


def _tc_final(agg, h, wr, b, deg, bm=512):
    m, k = h.shape
    S = agg.shape[0]
    dout = wr.shape[1]
    return pl.pallas_call(
        functools.partial(_final_body, S),
        grid=(m // bm,),
        in_specs=[
            pl.BlockSpec((bm, 1), lambda i: (i, 0)),
            pl.BlockSpec((S, bm, SLAB), lambda i: (0, i, 0)),
            pl.BlockSpec((bm, k), lambda i: (i, 0)),
            pl.BlockSpec((k, dout), lambda i: (0, 0)),
            pl.BlockSpec((1, dout), lambda i: (0, 0)),
        ],
        out_specs=pl.BlockSpec((bm, dout), lambda i: (i, 0)),
        out_shape=jax.ShapeDtypeStruct((m, dout), jnp.float32),
    )(deg, agg, h, wr, b)


def kernel(x, edge_index, W1_l, b1, W1_r, W2_l, b2, W2_r, W3_l, b3, W3_r):
    n = x.shape[0]
    e = edge_index.shape[1]
    x_pad = jnp.concatenate(
        [x, jnp.zeros((M_PAD - n, x.shape[1]), x.dtype)], axis=0)
    src = edge_index[0].astype(jnp.int32)
    dst = edge_index[1].astype(jnp.int32)
    src3 = jnp.concatenate(
        [src, jnp.zeros((E_PAD - e,), jnp.int32)]).reshape(NS, NCHUNK, CH)
    dst3 = jnp.concatenate(
        [dst, jnp.full((E_PAD - e,), n, jnp.int32)]).reshape(NS, NCHUNK, CH)

    deg = _sc_degree(dst3)[:, :1]
    agg1 = _sc_aggregate(x_pad, src3, dst3, S=x.shape[1] // SLAB)
    h1 = _tc_fused(agg1, x_pad, W1_l, W1_r, b1.reshape(1, -1), deg)
    agg2 = _sc_aggregate(h1, src3, dst3, S=h1.shape[1] // SLAB)
    h2 = _tc_fused(agg2, h1, W2_l, W2_r, b2.reshape(1, -1), deg)
    p = _tc_matmul(h2, W3_l)
    agg3 = _sc_aggregate(p, src3, dst3, S=p.shape[1] // SLAB)
    out = _tc_final(agg3, h2, W3_r, b3.reshape(1, -1), deg)
    return out[:n]


# speedup vs baseline: 2.7661x; 1.2761x over previous
import functools

import jax
import jax.numpy as jnp
from jax import lax
from jax.experimental import pallas as pl
from jax.experimental.pallas import tpu as pltpu
from jax.experimental.pallas import tpu_sc as plsc

N_NODES = 10000
N_EDGES = 160000
M_PAD = 10240
NS = 16
NC = 2
CH = 128
NCHUNK = 80
E_PAD = NS * NCHUNK * CH
RPT = M_PAD // NS
ZROWS = 320
SLAB = 64
NBUF = 4

_MESH = dict(core_axis_name="c", subcore_axis_name="s",
             num_cores=NC, num_subcores=NS)


def _fill(ref, rows, value):
    def _f(i, carry):
        for j in range(SLAB // 16):
            ref[i, pl.ds(j * 16, 16)] = jnp.full((16,), value, jnp.float32)
        return carry
    lax.fori_loop(0, rows, _f, 0)


def _deg_body(dst3_ref, deg_out, dstb, onesv, zerov, degacc):
    c = lax.axis_index("c")
    sid = lax.axis_index("s")
    pltpu.sync_copy(dst3_ref.at[sid], dstb)
    _fill(onesv, CH, 1.0)
    _fill(zerov, ZROWS, 0.0)
    for z in range(RPT // ZROWS):
        pltpu.sync_copy(zerov, degacc.at[pl.ds(sid * RPT + z * ZROWS, ZROWS)])
    plsc.subcore_barrier()

    def _edge(j, carry):
        pltpu.sync_copy(onesv, degacc.at[dstb.at[j]], add=True)
        return carry
    lax.fori_loop(0, NCHUNK, _edge, 0)
    plsc.subcore_barrier()

    @pl.when(c == 0)
    def _():
        pltpu.sync_copy(degacc.at[pl.ds(sid * RPT, RPT)],
                        deg_out.at[pl.ds(sid * RPT, RPT)])


def _sc_degree(dst3):
    call = pl.kernel(
        _deg_body,
        out_type=jax.ShapeDtypeStruct((M_PAD, SLAB), jnp.float32),
        mesh=plsc.VectorSubcoreMesh(**_MESH),
        compiler_params=pltpu.CompilerParams(use_tc_tiling_on_sc=False),
        scratch_types=(
            pltpu.VMEM((NCHUNK, CH), jnp.int32),
            pltpu.VMEM((CH, SLAB), jnp.float32),
            pltpu.VMEM((ZROWS, SLAB), jnp.float32),
            pltpu.VMEM_SHARED((M_PAD, SLAB), jnp.float32),
        ),
    )
    return call(dst3)


def _agg_body(S, h_ref, src3_ref, dst3_ref, agg_out,
              srcb, dstb, idxb, rowb, zerov, acc, sems):
    c = lax.axis_index("c")
    sid = lax.axis_index("s")
    s_half = S // NC

    pltpu.sync_copy(src3_ref.at[sid], srcb)
    pltpu.sync_copy(dst3_ref.at[sid], dstb)
    _fill(zerov, ZROWS, 0.0)

    for js in range(s_half):
        s_glob = c * s_half + js
        for z in range(RPT // ZROWS):
            pltpu.sync_copy(zerov,
                            acc.at[pl.ds(sid * RPT + z * ZROWS, ZROWS)])
        plsc.subcore_barrier()

        def _ix(i, carry):
            for j in range(CH // 16):
                idxb[i, pl.ds(j * 16, 16)] = (
                    srcb[i, pl.ds(j * 16, 16)] * S + s_glob)
            return carry
        lax.fori_loop(0, NCHUNK, _ix, 0)

        for b in range(NBUF):
            pltpu.async_copy(h_ref.at[idxb.at[b]], rowb.at[b], sems.at[b])

        def _edge(g, carry):
            for b in range(NBUF):
                j = g * NBUF + b
                pltpu.make_async_copy(
                    h_ref.at[idxb.at[j]], rowb.at[b], sems.at[b]).wait()
                pltpu.sync_copy(rowb.at[b], acc.at[dstb.at[j]], add=True)
                jn = j + NBUF

                @pl.when(jn < NCHUNK)
                def _():
                    pltpu.async_copy(
                        h_ref.at[idxb.at[jn]], rowb.at[b], sems.at[b])
            return carry
        lax.fori_loop(0, NCHUNK // NBUF, _edge, 0)
        plsc.subcore_barrier()

        pltpu.sync_copy(
            acc.at[pl.ds(sid * RPT, RPT)],
            agg_out.at[s_glob, pl.ds(sid * RPT, RPT)])
        if js != s_half - 1:
            plsc.subcore_barrier()


def _sc_aggregate(h, src3, dst3, S):
    call = pl.kernel(
        functools.partial(_agg_body, S),
        out_type=jax.ShapeDtypeStruct((S, M_PAD, SLAB), jnp.float32),
        mesh=plsc.VectorSubcoreMesh(**_MESH),
        compiler_params=pltpu.CompilerParams(use_tc_tiling_on_sc=False),
        scratch_types=(
            pltpu.VMEM((NCHUNK, CH), jnp.int32),
            pltpu.VMEM((NCHUNK, CH), jnp.int32),
            pltpu.VMEM((NCHUNK, CH), jnp.int32),
            pltpu.VMEM((NBUF, CH, SLAB), jnp.float32),
            pltpu.VMEM((ZROWS, SLAB), jnp.float32),
            pltpu.VMEM_SHARED((M_PAD, SLAB), jnp.float32),
            pltpu.SemaphoreType.DMA((NBUF,)),
        ),
    )
    return call(h.reshape(M_PAD * S, SLAB), src3, dst3)


def _fused_body(deg_ref, agg_ref, x_ref, wl_ref, wr_ref, b_ref, out_ref,
                acc_ref):
    s = pl.program_id(2)
    ns = pl.num_programs(2)

    @pl.when(s == 0)
    def _():
        acc_ref[...] = jnp.zeros_like(acc_ref)

    acc_ref[...] += jnp.dot(agg_ref[0], wl_ref[0],
                            preferred_element_type=jnp.float32)

    @pl.when(s == ns - 1)
    def _():
        scale = 1.0 / jnp.maximum(deg_ref[...], 1.0)
        acc = acc_ref[...] * scale + jnp.dot(
            x_ref[...], wr_ref[...], preferred_element_type=jnp.float32)
        out_ref[...] = jnp.maximum(acc + b_ref[...], 0.0)


def _tc_fused(agg, x, wl, wr, b, deg, bm=512, bn=512):
    m, k = x.shape
    S = agg.shape[0]
    dout = wl.shape[1]
    bn = min(bn, dout)
    wl3 = wl.reshape(S, SLAB, dout)
    grid = (m // bm, dout // bn, S)
    return pl.pallas_call(
        _fused_body,
        grid=grid,
        in_specs=[
            pl.BlockSpec((bm, 1), lambda i, j, s: (i, 0)),
            pl.BlockSpec((1, bm, SLAB), lambda i, j, s: (s, i, 0)),
            pl.BlockSpec((bm, k), lambda i, j, s: (i, 0)),
            pl.BlockSpec((1, SLAB, bn), lambda i, j, s: (s, 0, j)),
            pl.BlockSpec((k, bn), lambda i, j, s: (0, j)),
            pl.BlockSpec((1, bn), lambda i, j, s: (0, j)),
        ],
        out_specs=pl.BlockSpec((bm, bn), lambda i, j, s: (i, j)),
        out_shape=jax.ShapeDtypeStruct((m, dout), jnp.float32),
        scratch_shapes=[pltpu.VMEM((bm, bn), jnp.float32)],
    )(deg, agg, x, wl3, wr, b)


def _mm_body(a_ref, w_ref, o_ref):
    o_ref[...] = jnp.dot(a_ref[...], w_ref[...],
                         preferred_element_type=jnp.float32)


def _tc_matmul(a, w, bm=512):
    m, k = a.shape
    dout = w.shape[1]
    return pl.pallas_call(
        _mm_body,
        grid=(m // bm,),
        in_specs=[
            pl.BlockSpec((bm, k), lambda i: (i, 0)),
            pl.BlockSpec((k, dout), lambda i: (0, 0)),
        ],
        out_specs=pl.BlockSpec((bm, dout), lambda i: (i, 0)),
        out_shape=jax.ShapeDtypeStruct((m, dout), jnp.float32),
    )(a, w)


def _final_body(S, deg_ref, agg_ref, h_ref, wr_ref, b_ref, out_ref):
    scale = 1.0 / jnp.maximum(deg_ref[...], 1.0)
    agg = jnp.concatenate([agg_ref[s] for s in range(S)], axis=1)
    acc = agg * scale + jnp.dot(h_ref[...], wr_ref[...],
                                preferred_element_type=jnp.float32)
    out_ref[...] = jnp.maximum(acc + b_ref[...], 0.0)


def _tc_final(agg, h, wr, b, deg, bm=512):
    m, k = h.shape
    S = agg.shape[0]
    dout = wr.shape[1]
    return pl.pallas_call(
        functools.partial(_final_body, S),
        grid=(m // bm,),
        in_specs=[
            pl.BlockSpec((bm, 1), lambda i: (i, 0)),
            pl.BlockSpec((S, bm, SLAB), lambda i: (0, i, 0)),
            pl.BlockSpec((bm, k), lambda i: (i, 0)),
            pl.BlockSpec((k, dout), lambda i: (0, 0)),
            pl.BlockSpec((1, dout), lambda i: (0, 0)),
        ],
        out_specs=pl.BlockSpec((bm, dout), lambda i: (i, 0)),
        out_shape=jax.ShapeDtypeStruct((m, dout), jnp.float32),
    )(deg, agg, h, wr, b)


def kernel(x, edge_index, W1_l, b1, W1_r, W2_l, b2, W2_r, W3_l, b3, W3_r):
    n = x.shape[0]
    e = edge_index.shape[1]
    x_pad = jnp.concatenate(
        [x, jnp.zeros((M_PAD - n, x.shape[1]), x.dtype)], axis=0)
    src = edge_index[0].astype(jnp.int32)
    dst = edge_index[1].astype(jnp.int32)
    src3 = jnp.concatenate(
        [src, jnp.zeros((E_PAD - e,), jnp.int32)]).reshape(NS, NCHUNK, CH)
    dst3 = jnp.concatenate(
        [dst, jnp.full((E_PAD - e,), n, jnp.int32)]).reshape(NS, NCHUNK, CH)

    deg = _sc_degree(dst3)[:, :1]
    agg1 = _sc_aggregate(x_pad, src3, dst3, S=x.shape[1] // SLAB)
    h1 = _tc_fused(agg1, x_pad, W1_l, W1_r, b1.reshape(1, -1), deg)
    agg2 = _sc_aggregate(h1, src3, dst3, S=h1.shape[1] // SLAB)
    h2 = _tc_fused(agg2, h1, W2_l, W2_r, b2.reshape(1, -1), deg)
    p = _tc_matmul(h2, W3_l)
    agg3 = _sc_aggregate(p, src3, dst3, S=p.shape[1] // SLAB)
    out = _tc_final(agg3, h2, W3_r, b3.reshape(1, -1), deg)
    return out[:n]
